# baseline (speedup 1.0000x reference)
import functools

import jax
import jax.numpy as jnp
from jax import lax
from jax.experimental import pallas as pl
from jax.experimental.pallas import tpu as pltpu
from jax.experimental.pallas import tpu_sc as plsc

_H = 128
_D = 64
_V = 1024
_NBUF = 4


def _pos_encoding(seq_len, d_model):
    pos = jnp.arange(seq_len, dtype=jnp.float32)[:, None]
    dim = jnp.arange(0, d_model, 2, dtype=jnp.float32)
    angle = pos / jnp.power(10000.0, dim / float(d_model))
    res = jnp.zeros((seq_len, d_model), dtype=jnp.float32)
    res = res.at[:, 0::2].set(jnp.sin(angle))
    res = res.at[:, 1::2].set(jnp.cos(angle))
    return res


def kernel(target, Wx, Wy):
    B, T, N, _ = target.shape
    K = B * N * T * 2
    P = 2 * T

    ws = jnp.concatenate([Wx, Wy], axis=0)
    j = (jnp.transpose(target, (0, 2, 1, 3)).reshape(-1, 2)
         + jnp.array([0, _V], jnp.int32)).reshape(-1)
    pe2 = _pos_encoding(T, _H).reshape(P, _D)

    info = plsc.get_sparse_core_info()
    nw = info.num_cores * info.num_subcores
    per_w = K // nw
    ch = 2 * P
    n_ch = per_w // ch

    mesh = plsc.VectorSubcoreMesh(core_axis_name="c", subcore_axis_name="s")

    @functools.partial(
        pl.kernel,
        out_type=jax.ShapeDtypeStruct((K, _D), jnp.float32),
        mesh=mesh,
        compiler_params=pltpu.CompilerParams(use_tc_tiling_on_sc=False),
        scratch_types=(
            [pltpu.VMEM((ch,), jnp.int32) for _ in range(_NBUF)]
            + [pltpu.VMEM((ch, _D), jnp.float32) for _ in range(_NBUF)]
            + [pltpu.VMEM((P, _D), jnp.float32)]
            + [pltpu.SemaphoreType.DMA for _ in range(2 * _NBUF)]
        ),
    )
    def emb_kernel(ws_hbm, j_hbm, pe_hbm, out_hbm, *refs):
        idx = refs[:_NBUF]
        rows = refs[_NBUF:2 * _NBUF]
        pe_v = refs[2 * _NBUF]
        gsem = refs[2 * _NBUF + 1:2 * _NBUF + 1 + _NBUF]
        wsem = refs[2 * _NBUF + 1 + _NBUF:]

        wid = lax.axis_index("s") * info.num_cores + lax.axis_index("c")
        base = wid * per_w
        pltpu.sync_copy(pe_hbm, pe_v)

        def gather_start(ci):
            p = ci % _NBUF
            pltpu.sync_copy(j_hbm.at[pl.ds(base + ci * ch, ch)], idx[p])
            return pltpu.async_copy(ws_hbm.at[idx[p]], rows[p], gsem[p])

        gdesc = {0: gather_start(0), 1: gather_start(1)}
        wdesc = {}
        for ci in range(n_ch):
            p = ci % _NBUF
            gdesc.pop(ci).wait()

            @pl.loop(0, P, unroll=4)
            def _row(r):
                for q in range(_D // 16):
                    sl = pl.ds(q * 16, 16)
                    pv = pe_v[r, sl]
                    rows[p][r, sl] = rows[p][r, sl] + pv
                    rows[p][r + P, sl] = rows[p][r + P, sl] + pv

            wdesc[ci] = pltpu.async_copy(
                rows[p], out_hbm.at[pl.ds(base + ci * ch, ch)], wsem[p])
            nxt = ci + 2
            if nxt < n_ch:
                if nxt - _NBUF >= 0:
                    wdesc.pop(nxt - _NBUF).wait()
                gdesc[nxt] = gather_start(nxt)
        for ci in sorted(wdesc):
            wdesc.pop(ci).wait()

    out = emb_kernel(ws, j, pe2)
    return out.reshape(B * N, T, _H)


# speedup vs baseline: 1.2599x; 1.2599x over previous
import functools

import jax
import jax.numpy as jnp
from jax import lax
from jax.experimental import pallas as pl
from jax.experimental.pallas import tpu as pltpu
from jax.experimental.pallas import tpu_sc as plsc

_H = 128
_D = 64
_V = 1024
_NBUF = 4
_LOOKAHEAD = 2


def _pos_encoding(seq_len, d_model):
    pos = jnp.arange(seq_len, dtype=jnp.float32)[:, None]
    dim = jnp.arange(0, d_model, 2, dtype=jnp.float32)
    angle = pos / jnp.power(10000.0, dim / float(d_model))
    res = jnp.zeros((seq_len, d_model), dtype=jnp.float32)
    res = res.at[:, 0::2].set(jnp.sin(angle))
    res = res.at[:, 1::2].set(jnp.cos(angle))
    return res


def kernel(target, Wx, Wy):
    B, T, N, _ = target.shape
    K = B * N * T * 2
    P = 2 * T
    slab_w = N * 2 * T

    ws = jnp.concatenate([Wx, Wy], axis=0)
    tgt_flat = target.reshape(-1)
    pe2 = _pos_encoding(T, _H).reshape(P, _D)

    info = plsc.get_sparse_core_info()
    nw = info.num_cores * info.num_subcores
    per_w = K // nw
    b_per_w = B // nw
    ch = 2 * P
    n_ch = per_w // ch
    ch_per_b = n_ch // b_per_w
    nvec = ch // 16

    mesh = plsc.VectorSubcoreMesh(core_axis_name="c", subcore_axis_name="s")

    @functools.partial(
        pl.kernel,
        out_type=jax.ShapeDtypeStruct((K, _D), jnp.float32),
        mesh=mesh,
        compiler_params=pltpu.CompilerParams(
            use_tc_tiling_on_sc=False, needs_layout_passes=False),
        scratch_types=(
            [pltpu.VMEM((ch,), jnp.int32) for _ in range(_NBUF)]
            + [pltpu.VMEM((ch, _D), jnp.float32) for _ in range(_NBUF)]
            + [pltpu.VMEM((P, _D), jnp.float32),
               pltpu.VMEM((slab_w,), jnp.int32),
               pltpu.VMEM((ch,), jnp.int32)]
            + [pltpu.SemaphoreType.DMA for _ in range(2 * _NBUF)]
        ),
    )
    def emb_kernel(ws_hbm, tgt_hbm, pe_hbm, out_hbm, *refs):
        idx = refs[:_NBUF]
        rows = refs[_NBUF:2 * _NBUF]
        pe_v, slab_v, pat_v = refs[2 * _NBUF:2 * _NBUF + 3]
        gsem = refs[2 * _NBUF + 3:2 * _NBUF + 3 + _NBUF]
        wsem = refs[2 * _NBUF + 3 + _NBUF:]

        wid = lax.axis_index("s") * info.num_cores + lax.axis_index("c")
        base = wid * per_w
        slab0 = wid * (b_per_w * slab_w)
        pltpu.sync_copy(pe_hbm, pe_v)

        iota = lax.iota(jnp.int32, 16)
        cadd = (iota & 1) * _V

        @pl.loop(0, nvec)
        def _pat(v):
            e = v * 16 + iota
            s = jnp.where(e >= P, 1, 0)
            r = e - s * P
            pat_v[pl.ds(v * 16, 16)] = (
                (r >> 1) * (2 * N) + s * 2 + (r & 1))

        def load_slab(b_loc):
            pltpu.sync_copy(
                tgt_hbm.at[pl.ds(slab0 + b_loc * slab_w, slab_w)], slab_v)

        def build_idx(ci):
            n0x2 = ((2 * ci) % N) * 2

            @pl.loop(0, nvec)
            def _bld(v):
                sl = pl.ds(v * 16, 16)
                off = pat_v[sl] + n0x2
                idx[ci % _NBUF][sl] = plsc.load_gather(slab_v, [off]) + cadd

        def gather_start(ci):
            p = ci % _NBUF
            return pltpu.async_copy(ws_hbm.at[idx[p]], rows[p], gsem[p])

        load_slab(0)
        gdesc, wdesc = {}, {}
        for ci in range(_LOOKAHEAD):
            build_idx(ci)
            gdesc[ci] = gather_start(ci)

        for ci in range(n_ch):
            p = ci % _NBUF
            gdesc.pop(ci).wait()

            @pl.loop(0, P, unroll=4)
            def _row(r):
                for q in range(_D // 16):
                    sl = pl.ds(q * 16, 16)
                    pv = pe_v[r, sl]
                    rows[p][r, sl] = rows[p][r, sl] + pv
                    rows[p][r + P, sl] = rows[p][r + P, sl] + pv

            wdesc[ci] = pltpu.async_copy(
                rows[p], out_hbm.at[pl.ds(base + ci * ch, ch)], wsem[p])
            nxt = ci + _LOOKAHEAD
            if nxt < n_ch:
                if nxt - _NBUF >= 0:
                    wdesc.pop(nxt - _NBUF).wait()
                if nxt % ch_per_b == 0:
                    load_slab(nxt // ch_per_b)
                build_idx(nxt)
                gdesc[nxt] = gather_start(nxt)
        for ci in sorted(wdesc):
            wdesc.pop(ci).wait()

    out = emb_kernel(ws, tgt_flat, pe2)
    return out.reshape(B * N, T, _H)


# speedup vs baseline: 1.3630x; 1.0819x over previous
import functools

import jax
import jax.numpy as jnp
from jax import lax
from jax.experimental import pallas as pl
from jax.experimental.pallas import tpu as pltpu
from jax.experimental.pallas import tpu_sc as plsc

_H = 128
_D = 64
_V = 1024
_NBUF = 2
_LOOKAHEAD = 2


def _pos_encoding(seq_len, d_model):
    pos = jnp.arange(seq_len, dtype=jnp.float32)[:, None]
    dim = jnp.arange(0, d_model, 2, dtype=jnp.float32)
    angle = pos / jnp.power(10000.0, dim / float(d_model))
    res = jnp.zeros((seq_len, d_model), dtype=jnp.float32)
    res = res.at[:, 0::2].set(jnp.sin(angle))
    res = res.at[:, 1::2].set(jnp.cos(angle))
    return res


def kernel(target, Wx, Wy):
    B, T, N, _ = target.shape
    K = B * N * T * 2
    P = 2 * T
    slab_w = N * 2 * T

    ws = jnp.concatenate([Wx, Wy], axis=0)
    tgt_flat = target.reshape(-1)
    pe2 = _pos_encoding(T, _H).reshape(P, _D)

    info = plsc.get_sparse_core_info()
    nw = info.num_cores * info.num_subcores
    per_w = K // nw
    b_per_w = B // nw
    ch = 2 * P
    n_ch = per_w // ch
    ch_per_b = n_ch // b_per_w
    nvec = ch // 16
    g_per_w = per_w // P

    mesh = plsc.VectorSubcoreMesh(core_axis_name="c", subcore_axis_name="s")

    @functools.partial(
        pl.kernel,
        out_type=jax.ShapeDtypeStruct((B * N, T, _H), jnp.float32),
        mesh=mesh,
        compiler_params=pltpu.CompilerParams(
            use_tc_tiling_on_sc=False, needs_layout_passes=False),
        scratch_types=(
            [pltpu.VMEM((ch,), jnp.int32) for _ in range(_NBUF)]
            + [pltpu.VMEM((ch, _D), jnp.float32) for _ in range(_NBUF)]
            + [pltpu.VMEM((2, T, _H), jnp.float32) for _ in range(_NBUF)]
            + [pltpu.VMEM((P, _D), jnp.float32),
               pltpu.VMEM((slab_w,), jnp.int32),
               pltpu.VMEM((ch,), jnp.int32)]
            + [pltpu.SemaphoreType.DMA for _ in range(2 * _NBUF)]
        ),
    )
    def emb_kernel(ws_hbm, tgt_hbm, pe_hbm, out_hbm, *refs):
        idx = refs[:_NBUF]
        gbuf = refs[_NBUF:2 * _NBUF]
        obuf = refs[2 * _NBUF:3 * _NBUF]
        pe_v, slab_v, pat_v = refs[3 * _NBUF:3 * _NBUF + 3]
        gsem = refs[3 * _NBUF + 3:3 * _NBUF + 3 + _NBUF]
        wsem = refs[3 * _NBUF + 3 + _NBUF:]

        wid = lax.axis_index("s") * info.num_cores + lax.axis_index("c")
        slab0 = wid * (b_per_w * slab_w)
        bn0 = wid * g_per_w
        pltpu.sync_copy(pe_hbm, pe_v)

        iota = lax.iota(jnp.int32, 16)
        cadd = (iota & 1) * _V

        @pl.loop(0, nvec)
        def _pat(v):
            e = v * 16 + iota
            s = jnp.where(e >= P, 1, 0)
            r = e - s * P
            pat_v[pl.ds(v * 16, 16)] = (
                (r >> 1) * (2 * N) + s * 2 + (r & 1))

        def load_slab(b_loc):
            pltpu.sync_copy(
                tgt_hbm.at[pl.ds(slab0 + b_loc * slab_w, slab_w)], slab_v)

        def build_idx(ci):
            n0x2 = ((2 * ci) % N) * 2

            @pl.loop(0, nvec)
            def _bld(v):
                sl = pl.ds(v * 16, 16)
                off = pat_v[sl] + n0x2
                idx[ci % _NBUF][sl] = plsc.load_gather(slab_v, [off]) + cadd

        def gather_start(ci):
            p = ci % _NBUF
            return pltpu.async_copy(ws_hbm.at[idx[p]], gbuf[p], gsem[p])

        load_slab(0)
        gdesc, wdesc = {}, {}
        for ci in range(_LOOKAHEAD):
            build_idx(ci)
            gdesc[ci] = gather_start(ci)

        for ci in range(n_ch):
            p = ci % _NBUF
            gdesc.pop(ci).wait()
            if ci - _NBUF in wdesc:
                wdesc.pop(ci - _NBUF).wait()

            @pl.loop(0, T)
            def _row(t):
                for c in range(2):
                    for q in range(_D // 16):
                        sl = pl.ds(q * 16, 16)
                        pv = pe_v[t * 2 + c, sl]
                        dst = pl.ds(c * _D + q * 16, 16)
                        obuf[p][0, t, dst] = gbuf[p][t * 2 + c, sl] + pv
                        obuf[p][1, t, dst] = gbuf[p][P + t * 2 + c, sl] + pv

            wdesc[ci] = pltpu.async_copy(
                obuf[p], out_hbm.at[pl.ds(bn0 + 2 * ci, 2)], wsem[p])
            nxt = ci + _LOOKAHEAD
            if nxt < n_ch:
                if nxt % ch_per_b == 0:
                    load_slab(nxt // ch_per_b)
                build_idx(nxt)
                gdesc[nxt] = gather_start(nxt)
        for ci in sorted(wdesc):
            wdesc.pop(ci).wait()

    return emb_kernel(ws, tgt_flat, pe2)


# speedup vs baseline: 1.4754x; 1.0825x over previous
import functools

import jax
import jax.numpy as jnp
from jax import lax
from jax.experimental import pallas as pl
from jax.experimental.pallas import tpu as pltpu
from jax.experimental.pallas import tpu_sc as plsc

_H = 128
_D = 64
_V = 1024
_NBUF = 2
_LOOKAHEAD = 2


def _pos_encoding(seq_len, d_model):
    pos = jnp.arange(seq_len, dtype=jnp.float32)[:, None]
    dim = jnp.arange(0, d_model, 2, dtype=jnp.float32)
    angle = pos / jnp.power(10000.0, dim / float(d_model))
    res = jnp.zeros((seq_len, d_model), dtype=jnp.float32)
    res = res.at[:, 0::2].set(jnp.sin(angle))
    res = res.at[:, 1::2].set(jnp.cos(angle))
    return res


def kernel(target, Wx, Wy):
    B, T, N, _ = target.shape
    K = B * N * T * 2
    P = 2 * T
    slab_w = N * 2 * T

    ws = jnp.concatenate([Wx, Wy], axis=0)
    pe2 = _pos_encoding(T, _H).reshape(P, _D)
    Tp = 104

    info = plsc.get_sparse_core_info()
    nw = info.num_cores * info.num_subcores
    per_w = K // nw
    b_per_w = B // nw
    ch = 2 * P
    n_ch = per_w // ch
    ch_per_b = n_ch // b_per_w
    nvec = ch // 16
    g_per_w = per_w // P

    mesh = plsc.VectorSubcoreMesh(core_axis_name="c", subcore_axis_name="s")

    @functools.partial(
        pl.kernel,
        out_type=jax.ShapeDtypeStruct((B * N, Tp, _H), jnp.float32),
        mesh=mesh,
        compiler_params=pltpu.CompilerParams(
            use_tc_tiling_on_sc=False, needs_layout_passes=False),
        scratch_types=(
            [pltpu.VMEM((ch,), jnp.int32) for _ in range(_NBUF)]
            + [pltpu.VMEM((ch, _D), jnp.float32) for _ in range(_NBUF)]
            + [pltpu.VMEM((2, T, _H), jnp.float32) for _ in range(_NBUF)]
            + [pltpu.VMEM((P, _D), jnp.float32),
               pltpu.VMEM((T, N, 2), jnp.int32),
               pltpu.VMEM((ch,), jnp.int32),
               pltpu.VMEM((ch,), jnp.int32)]
            + [pltpu.SemaphoreType.DMA for _ in range(3 * _NBUF)]
        ),
    )
    def emb_kernel(ws_hbm, tgt_hbm, pe_hbm, out_hbm, *refs):
        idx = refs[:_NBUF]
        gbuf = refs[_NBUF:2 * _NBUF]
        obuf = refs[2 * _NBUF:3 * _NBUF]
        pe_v, slab_v, patt_v, patn_v = refs[3 * _NBUF:3 * _NBUF + 4]
        gsem = refs[3 * _NBUF + 4:3 * _NBUF + 4 + _NBUF]
        wsem = refs[3 * _NBUF + 4 + _NBUF:]

        wid = lax.axis_index("s") * info.num_cores + lax.axis_index("c")
        b0 = wid * b_per_w
        bn0 = wid * g_per_w
        pltpu.sync_copy(pe_hbm, pe_v)

        iota = lax.iota(jnp.int32, 16)
        cadd = (iota & 1) * _V
        pcol = iota & 1

        @pl.loop(0, nvec)
        def _pat(v):
            e = v * 16 + iota
            s = jnp.where(e >= P, 1, 0)
            r = e - s * P
            patt_v[pl.ds(v * 16, 16)] = r >> 1
            patn_v[pl.ds(v * 16, 16)] = s

        def load_slab(b_loc):
            pltpu.sync_copy(tgt_hbm.at[b0 + b_loc], slab_v)

        def build_idx(ci):
            n0 = (2 * ci) % N

            @pl.loop(0, nvec)
            def _bld(v):
                sl = pl.ds(v * 16, 16)
                idx[ci % _NBUF][sl] = plsc.load_gather(
                    slab_v, [patt_v[sl], patn_v[sl] + n0, pcol]) + cadd

        def gather_start(ci):
            p = ci % _NBUF
            return pltpu.async_copy(ws_hbm.at[idx[p]], gbuf[p], gsem[p])

        load_slab(0)
        gdesc, wdesc = {}, {}
        for ci in range(_LOOKAHEAD):
            build_idx(ci)
            gdesc[ci] = gather_start(ci)

        for ci in range(n_ch):
            p = ci % _NBUF
            gdesc.pop(ci).wait()
            if ci - _NBUF in wdesc:
                for d in wdesc.pop(ci - _NBUF):
                    d.wait()

            @pl.loop(0, T)
            def _row(t):
                for c in range(2):
                    for q in range(_D // 16):
                        sl = pl.ds(q * 16, 16)
                        pv = pe_v[t * 2 + c, sl]
                        dst = pl.ds(c * _D + q * 16, 16)
                        obuf[p][0, t, dst] = gbuf[p][t * 2 + c, sl] + pv
                        obuf[p][1, t, dst] = gbuf[p][P + t * 2 + c, sl] + pv

            wdesc[ci] = [
                pltpu.async_copy(
                    obuf[p].at[s],
                    out_hbm.at[bn0 + 2 * ci + s, pl.ds(0, T)],
                    wsem[2 * p + s])
                for s in range(2)]
            nxt = ci + _LOOKAHEAD
            if nxt < n_ch:
                if nxt % ch_per_b == 0:
                    load_slab(nxt // ch_per_b)
                build_idx(nxt)
                gdesc[nxt] = gather_start(nxt)
        for ci in sorted(wdesc):
            for d in wdesc.pop(ci):
                d.wait()

    return emb_kernel(ws, target, pe2)[:, :T, :]
